# baseline (speedup 1.0000x reference)
import functools

import jax
import jax.numpy as jnp
from jax import lax
from jax.experimental import pallas as pl
from jax.experimental.pallas import tpu as pltpu
from jax.experimental.pallas import tpu_sc as plsc

_LANES = 16


@functools.lru_cache(maxsize=None)
def _make_sc_gather(n, v, d):
    info = plsc.get_sparse_core_info()
    nc, ns = info.num_cores, info.num_subcores
    nw = nc * ns
    per_w = n // nw
    chunk = 8
    nbuf = 8
    depth = 4
    n_chunks = per_w // chunk
    assert per_w % chunk == 0 and n_chunks % nbuf == 0

    mesh = plsc.VectorSubcoreMesh(core_axis_name="c", subcore_axis_name="s")

    @functools.partial(
        pl.kernel,
        out_type=jax.ShapeDtypeStruct((n, d), jnp.float32),
        mesh=mesh,
        scratch_types=[
            pltpu.VMEM((per_w,), jnp.int32),
            pltpu.VMEM((nbuf * chunk, d), jnp.float32),
            pltpu.SemaphoreType.DMA((nbuf,)),
            pltpu.SemaphoreType.DMA((nbuf,)),
        ],
    )
    def gather_kernel(idx_hbm, table_hbm, out_hbm, idx_v, rows_v, in_sems, out_sems):
        wid = lax.axis_index("s") * nc + lax.axis_index("c")
        base = wid * per_w
        pltpu.sync_copy(idx_hbm.at[pl.ds(base, per_w)], idx_v)

        def clamp_body(i, carry):
            sl = pl.ds(i * _LANES, _LANES)
            x = idx_v[sl]
            idx_v[sl] = jnp.minimum(jnp.maximum(x, 0), v - 1)
            return carry

        lax.fori_loop(0, per_w // _LANES, clamp_body, 0, unroll=4)

        def gather_chunk(j, slot):
            idx_slice = idx_v.at[pl.ds(j * chunk, chunk)]
            dst = rows_v.at[pl.ds(slot * chunk, chunk)]
            return pltpu.async_copy(table_hbm.at[idx_slice], dst, in_sems.at[slot])

        def wait_in(slot):
            pltpu.make_async_copy(
                table_hbm.at[idx_v.at[pl.ds(0, chunk)]],
                rows_v.at[pl.ds(slot * chunk, chunk)],
                in_sems.at[slot],
            ).wait()

        def wait_out(slot):
            pltpu.make_async_copy(
                rows_v.at[pl.ds(slot * chunk, chunk)],
                out_hbm.at[pl.ds(base, chunk)],
                out_sems.at[slot],
            ).wait()

        for g in range(depth):
            gather_chunk(g, g)

        def ring_body(k, carry):
            for s in range(nbuf):
                j = k * nbuf + s
                wait_in(s)
                s2 = (s + depth) % nbuf

                @pl.when(j + depth - nbuf >= 0)
                def _():
                    wait_out(s2)

                @pl.when(j + depth < n_chunks)
                def _():
                    gather_chunk(j + depth, s2)

                pltpu.async_copy(
                    rows_v.at[pl.ds(s * chunk, chunk)],
                    out_hbm.at[pl.ds(base + j * chunk, chunk)],
                    out_sems.at[s],
                )
            return carry

        lax.fori_loop(0, n_chunks // nbuf, ring_body, 0)
        for m in range(n_chunks - (nbuf - depth), n_chunks):
            wait_out(m % nbuf)

    return gather_kernel


@functools.lru_cache(maxsize=None)
def _make_tc_gather(m, v, d, br):
    def body(idx_ref, table_ref, out_ref):
        i = pl.program_id(0)

        def row(r, carry):
            idx = jnp.clip(idx_ref[i * br + r], 0, v - 1)
            out_ref[pl.ds(r, 1), :] = table_ref[pl.ds(idx, 1), :]
            return carry

        lax.fori_loop(0, br, row, 0, unroll=8)

    grid_spec = pltpu.PrefetchScalarGridSpec(
        num_scalar_prefetch=1,
        grid=(m // br,),
        in_specs=[pl.BlockSpec((v, d), lambda i, idx_ref: (0, 0))],
        out_specs=pl.BlockSpec((br, d), lambda i, idx_ref: (i, 0)),
    )
    return pl.pallas_call(
        body,
        grid_spec=grid_spec,
        out_shape=jax.ShapeDtypeStruct((m, d), jnp.float32),
        compiler_params=pltpu.CompilerParams(
            dimension_semantics=("arbitrary",),
        ),
    )


def kernel(position_ids, table):
    b, s = position_ids.shape
    v, d = table.shape
    n = b * s
    n_sc = 20480
    idx_flat = position_ids.reshape(n).astype(jnp.int32)
    out_sc = _make_sc_gather(n_sc, v, d)(idx_flat[:n_sc], table)
    out_tc = _make_tc_gather(n - n_sc, v, d, 256)(idx_flat[n_sc:], table)
    out = jnp.concatenate([out_sc, out_tc], axis=0)
    return out.reshape(b, s, d)


# speedup vs baseline: 1.8813x; 1.8813x over previous
import functools

import jax
import jax.numpy as jnp
from jax import lax
from jax.experimental import pallas as pl
from jax.experimental.pallas import tpu as pltpu
from jax.experimental.pallas import tpu_sc as plsc

_LANES = 16


@functools.lru_cache(maxsize=None)
def _make_gather(n, v, d):
    info = plsc.get_sparse_core_info()
    nc, ns = info.num_cores, info.num_subcores
    nw = nc * ns
    assert n % nw == 0
    per_w = n // nw
    chunk = 8
    nbuf = 8
    depth = 4
    assert per_w % chunk == 0
    n_chunks = per_w // chunk
    assert n_chunks % nbuf == 0 and depth < nbuf

    mesh = plsc.VectorSubcoreMesh(core_axis_name="c", subcore_axis_name="s")

    @functools.partial(
        pl.kernel,
        out_type=jax.ShapeDtypeStruct((n, d), jnp.float32),
        mesh=mesh,
        scratch_types=[
            pltpu.VMEM((per_w,), jnp.int32),
            pltpu.VMEM((nbuf * chunk, d), jnp.float32),
            pltpu.SemaphoreType.DMA((nbuf,)),
            pltpu.SemaphoreType.DMA((nbuf,)),
        ],
    )
    def gather_kernel(idx_hbm, table_hbm, out_hbm, idx_v, rows_v, in_sems, out_sems):
        wid = lax.axis_index("s") * nc + lax.axis_index("c")
        base = wid * per_w
        pltpu.sync_copy(idx_hbm.at[pl.ds(base, per_w)], idx_v)

        def clamp_body(i, carry):
            sl = pl.ds(i * _LANES, _LANES)
            x = idx_v[sl]
            idx_v[sl] = jnp.minimum(jnp.maximum(x, 0), v - 1)
            return carry

        lax.fori_loop(0, per_w // _LANES, clamp_body, 0, unroll=4)

        def gather_chunk(j, slot):
            idx_slice = idx_v.at[pl.ds(j * chunk, chunk)]
            dst = rows_v.at[pl.ds(slot * chunk, chunk)]
            return pltpu.async_copy(table_hbm.at[idx_slice], dst, in_sems.at[slot])

        def wait_in(slot):
            pltpu.make_async_copy(
                table_hbm.at[idx_v.at[pl.ds(0, chunk)]],
                rows_v.at[pl.ds(slot * chunk, chunk)],
                in_sems.at[slot],
            ).wait()

        def wait_out(slot):
            pltpu.make_async_copy(
                rows_v.at[pl.ds(slot * chunk, chunk)],
                out_hbm.at[pl.ds(base, chunk)],
                out_sems.at[slot],
            ).wait()

        for g in range(depth):
            gather_chunk(g, g)

        def ring_body(k, carry):
            for s in range(nbuf):
                j = k * nbuf + s
                wait_in(s)
                s2 = (s + depth) % nbuf

                @pl.when(j + depth - nbuf >= 0)
                def _():
                    wait_out(s2)

                @pl.when(j + depth < n_chunks)
                def _():
                    gather_chunk(j + depth, s2)

                pltpu.async_copy(
                    rows_v.at[pl.ds(s * chunk, chunk)],
                    out_hbm.at[pl.ds(base + j * chunk, chunk)],
                    out_sems.at[s],
                )
            return carry

        lax.fori_loop(0, n_chunks // nbuf, ring_body, 0)
        for m in range(n_chunks - (nbuf - depth), n_chunks):
            wait_out(m % nbuf)

    return gather_kernel


def kernel(position_ids, table):
    b, s = position_ids.shape
    v, d = table.shape
    n = b * s
    idx_flat = position_ids.reshape(n).astype(jnp.int32)
    out = _make_gather(n, v, d)(idx_flat, table)
    return out.reshape(b, s, d)
